# baseline (speedup 1.0000x reference)
import functools

import jax
import jax.numpy as jnp
from jax import lax
from jax.experimental import pallas as pl
from jax.experimental.pallas import tpu as pltpu
from jax.experimental.pallas import tpu_sc as plsc

N = 10000
E = 320000
D = 128
NP = 10240
BR = 128
NBLK = NP // BR
NSC = 2
NSUB = 16
NW = NSC * NSUB
ZPT = NP // NSUB
EPT = E // NW

CH = 80
NCH = EPT // CH
CH2 = 40
GRP = 10
NG = EPT // (CH2 * GRP)
NB = 4

_mesh = plsc.VectorSubcoreMesh(core_axis_name="c", subcore_axis_name="s")
_sc_params = pltpu.CompilerParams(needs_layout_passes=False)


@functools.partial(
    pl.kernel,
    out_type=jax.ShapeDtypeStruct((NSC, NP), jnp.float32),
    mesh=_mesh,
    scratch_types=[
        pltpu.VMEM((NCH, CH), jnp.int32),
        pltpu.VMEM((CH,), jnp.float32),
        pltpu.VMEM_SHARED((NP,), jnp.float32),
        pltpu.SemaphoreType.DMA,
    ],
    compiler_params=_sc_params,
)
def _sc_count(dst3_hbm, z1_hbm, out_hbm, didx, ones, acc, ssem):
    cid = lax.axis_index("c")
    sid = lax.axis_index("s")
    wid = cid * NSUB + sid
    for k in range(CH // 16):
        ones[pl.ds(16 * k, 16)] = jnp.full((16,), 1.0, jnp.float32)
    pltpu.sync_copy(z1_hbm.at[pl.ds(sid * ZPT, ZPT)], acc.at[pl.ds(sid * ZPT, ZPT)])
    pltpu.sync_copy(dst3_hbm.at[wid], didx)
    plsc.subcore_barrier()

    @pl.loop(0, NCH)
    def _(j):
        pltpu.async_copy(ones, acc.at[didx.at[j]], ssem, add=True)

    @pl.loop(0, NCH)
    def _(j):
        pltpu.make_async_copy(ones, acc.at[didx.at[j]], ssem).wait()

    plsc.subcore_barrier()
    pltpu.sync_copy(acc.at[pl.ds(sid * ZPT, ZPT)],
                    out_hbm.at[cid, pl.ds(sid * ZPT, ZPT)])


@functools.partial(
    pl.kernel,
    out_type=jax.ShapeDtypeStruct((NSC, NP, D), jnp.float32),
    mesh=_mesh,
    scratch_types=[
        pltpu.VMEM((2 * GRP, CH2), jnp.int32),
        pltpu.VMEM((CH2, D), jnp.float32),
        pltpu.VMEM((CH2, D), jnp.float32),
        pltpu.VMEM((CH2, D), jnp.float32),
        pltpu.VMEM((CH2, D), jnp.float32),
        pltpu.VMEM_SHARED((NP, D), jnp.float32),
    ] + [pltpu.SemaphoreType.DMA] * 8,
    compiler_params=_sc_params,
)
def _sc_agg(sd4_hbm, g_hbm, z2_hbm, out_hbm,
            idxg, rows0, rows1, rows2, rows3, acc,
            gsem0, gsem1, gsem2, gsem3, ssem0, ssem1, ssem2, ssem3):
    rows = (rows0, rows1, rows2, rows3)
    gsem = (gsem0, gsem1, gsem2, gsem3)
    ssem = (ssem0, ssem1, ssem2, ssem3)
    cid = lax.axis_index("c")
    sid = lax.axis_index("s")
    wid = cid * NSUB + sid
    pltpu.sync_copy(z2_hbm.at[pl.ds(sid * ZPT, ZPT)], acc.at[pl.ds(sid * ZPT, ZPT)])
    plsc.subcore_barrier()

    @pl.loop(0, NG)
    def _(g):
        pltpu.sync_copy(sd4_hbm.at[wid, g], idxg)
        gd = {}
        sd = {}
        for i in range(NB):
            gd[i] = pltpu.async_copy(g_hbm.at[idxg.at[i]], rows[i % NB],
                                     gsem[i % NB])
        for i in range(GRP):
            gd[i].wait()
            sd[i] = pltpu.async_copy(rows[i % NB], acc.at[idxg.at[GRP + i]],
                                     ssem[i % NB], add=True)
            if i + NB < GRP:
                sd[i].wait()
                gd[i + NB] = pltpu.async_copy(g_hbm.at[idxg.at[i + NB]],
                                              rows[i % NB], gsem[i % NB])
        for i in range(GRP - NB, GRP):
            sd[i].wait()

    plsc.subcore_barrier()
    pltpu.sync_copy(acc.at[pl.ds(sid * ZPT, ZPT)],
                    out_hbm.at[cid, pl.ds(sid * ZPT, ZPT)])


@functools.partial(
    pl.kernel,
    out_type=(jax.ShapeDtypeStruct((NSC, NP), jnp.float32),
              jax.ShapeDtypeStruct((E,), jnp.float32)),
    mesh=_mesh,
    scratch_types=[
        pltpu.VMEM((NCH, CH), jnp.int32),
        pltpu.VMEM((NCH, CH), jnp.int32),
        pltpu.VMEM((EPT,), jnp.float32),
        pltpu.VMEM((NP,), jnp.float32),
        pltpu.VMEM((NP,), jnp.float32),
        pltpu.VMEM((16,), jnp.float32),
        pltpu.VMEM((16,), jnp.float32),
        pltpu.VMEM_SHARED((NP,), jnp.float32),
        pltpu.SemaphoreType.DMA,
    ],
    compiler_params=_sc_params,
)
def _sc_eegen(src3_hbm, dst3_hbm, als_hbm, ald_hbm, ms_hbm, md_hbm, z1_hbm,
              dp_hbm, ee_hbm,
              sidx, didx, eeall, als_v, ald_v, msv, mdv, dacc, ssem):
    cid = lax.axis_index("c")
    sid = lax.axis_index("s")
    wid = cid * NSUB + sid
    pltpu.sync_copy(als_hbm, als_v)
    pltpu.sync_copy(ald_hbm, ald_v)
    pltpu.sync_copy(ms_hbm.at[0, pl.ds(0, 16)], msv)
    pltpu.sync_copy(md_hbm.at[0, pl.ds(0, 16)], mdv)
    pltpu.sync_copy(z1_hbm.at[pl.ds(sid * ZPT, ZPT)], dacc.at[pl.ds(sid * ZPT, ZPT)])
    pltpu.sync_copy(src3_hbm.at[wid], sidx)
    pltpu.sync_copy(dst3_hbm.at[wid], didx)
    plsc.subcore_barrier()
    mvec = jnp.maximum(msv[...] + mdv[...], 0.0)

    @pl.loop(0, NCH)
    def _(j):
        @pl.loop(0, CH // 16)
        def _(k):
            s16 = sidx[j, pl.ds(16 * k, 16)]
            d16 = didx[j, pl.ds(16 * k, 16)]
            e = plsc.load_gather(als_v, [s16]) + plsc.load_gather(ald_v, [d16])
            e = jnp.where(e > 0, e, 0.2 * e)
            eeall[pl.ds(j * CH + 16 * k, 16)] = jnp.exp(e - mvec)

        pltpu.async_copy(eeall.at[pl.ds(j * CH, CH)], dacc.at[didx.at[j]],
                         ssem, add=True)

    @pl.loop(0, NCH)
    def _(j):
        pltpu.make_async_copy(eeall.at[pl.ds(j * CH, CH)], dacc.at[didx.at[j]],
                              ssem).wait()

    plsc.subcore_barrier()
    pltpu.sync_copy(dacc.at[pl.ds(sid * ZPT, ZPT)],
                    dp_hbm.at[cid, pl.ds(sid * ZPT, ZPT)])
    pltpu.sync_copy(eeall, ee_hbm.at[pl.ds(wid * EPT, EPT)])


@functools.partial(
    pl.kernel,
    out_type=jax.ShapeDtypeStruct((NSC, NP, D), jnp.float32),
    mesh=_mesh,
    scratch_types=[
        pltpu.VMEM((3 * GRP, CH2), jnp.int32),
        pltpu.VMEM((CH2, D), jnp.float32),
        pltpu.VMEM((CH2, D), jnp.float32),
        pltpu.VMEM((CH2, D), jnp.float32),
        pltpu.VMEM((CH2, D), jnp.float32),
        pltpu.VMEM_SHARED((NP, D), jnp.float32),
    ] + [pltpu.SemaphoreType.DMA] * 8,
    compiler_params=_sc_params,
)
def _sc_gatagg(sde4_hbm, h2_hbm, z2_hbm, out_hbm,
               idxg, rows0, rows1, rows2, rows3, acc,
               gsem0, gsem1, gsem2, gsem3, ssem0, ssem1, ssem2, ssem3):
    rows = (rows0, rows1, rows2, rows3)
    gsem = (gsem0, gsem1, gsem2, gsem3)
    ssem = (ssem0, ssem1, ssem2, ssem3)
    cid = lax.axis_index("c")
    sid = lax.axis_index("s")
    wid = cid * NSUB + sid
    pltpu.sync_copy(z2_hbm.at[pl.ds(sid * ZPT, ZPT)], acc.at[pl.ds(sid * ZPT, ZPT)])
    plsc.subcore_barrier()

    def scale(i):
        i16 = jnp.full((16,), 2 * GRP + i, jnp.int32)
        one16 = jnp.full((16,), 1, jnp.int32)

        @pl.loop(0, CH2)
        def _(r):
            w = plsc.bitcast(plsc.load_gather(idxg, [i16, one16 * r]),
                             jnp.float32)
            for c in range(D // 16):
                rows[i % NB][r, pl.ds(16 * c, 16)] = (
                    rows[i % NB][r, pl.ds(16 * c, 16)] * w)

    @pl.loop(0, NG)
    def _(g):
        pltpu.sync_copy(sde4_hbm.at[wid, g], idxg)
        gd = {}
        sd = {}
        for i in range(NB):
            gd[i] = pltpu.async_copy(h2_hbm.at[idxg.at[i]], rows[i % NB],
                                     gsem[i % NB])
        for i in range(GRP):
            gd[i].wait()
            scale(i)
            sd[i] = pltpu.async_copy(rows[i % NB], acc.at[idxg.at[GRP + i]],
                                     ssem[i % NB], add=True)
            if i + NB < GRP:
                sd[i].wait()
                gd[i + NB] = pltpu.async_copy(h2_hbm.at[idxg.at[i + NB]],
                                              rows[i % NB], gsem[i % NB])
        for i in range(GRP - NB, GRP):
            sd[i].wait()

    plsc.subcore_barrier()
    pltpu.sync_copy(acc.at[pl.ds(sid * ZPT, ZPT)],
                    out_hbm.at[cid, pl.ds(sid * ZPT, ZPT)])


def _k0_body(x_ref, w_ref, h_ref):
    h_ref[...] = jnp.dot(x_ref[...], w_ref[...],
                         preferred_element_type=jnp.float32)


def _k1_body(h_ref, cnt_ref, g_ref, dinv_ref):
    cnt = cnt_ref[0, 0, 0, :] + cnt_ref[1, 0, 0, :]
    dinv = lax.rsqrt(cnt + 1.0)
    g_ref[...] = h_ref[...] * dinv.reshape(BR, 1)
    dinv_ref[...] = dinv.reshape(1, 1, BR)


def _k2_body(p_ref, g_ref, dinv_ref, b1_ref, w2_ref, as_ref, ad_ref,
             h2_ref, als_ref, ald_ref, ms_ref, md_ref):
    i = pl.program_id(0)
    dinv = dinv_ref[0, 0, :].reshape(BR, 1)
    sig = p_ref[0] + p_ref[1] + g_ref[...]
    h1 = jnp.maximum(dinv * sig + b1_ref[...].reshape(1, D), 0.0)
    h2 = jnp.dot(h1, w2_ref[...], preferred_element_type=jnp.float32)
    h2_ref[...] = h2
    als = jnp.sum(h2 * as_ref[...].reshape(1, D), axis=1)
    ald = jnp.sum(h2 * ad_ref[...].reshape(1, D), axis=1)
    als_ref[...] = als.reshape(1, 1, BR)
    ald_ref[...] = ald.reshape(1, 1, BR)
    bs = jnp.max(als)
    bd = jnp.max(ald)

    @pl.when(i == 0)
    def _():
        ms_ref[...] = jnp.full((8, 128), bs, jnp.float32)
        md_ref[...] = jnp.full((8, 128), bd, jnp.float32)

    @pl.when(i > 0)
    def _():
        ms_ref[...] = jnp.maximum(ms_ref[...], bs)
        md_ref[...] = jnp.maximum(md_ref[...], bd)


def _k4_body(np_ref, dp_ref, h2_ref, als_ref, ald_ref, ms_ref, md_ref,
             b2_ref, h3_ref):
    m = jnp.maximum(ms_ref[0, 0] + md_ref[0, 0], 0.0)
    al = als_ref[0, 0, :] + ald_ref[0, 0, :]
    el = jnp.where(al > 0, al, 0.2 * al)
    eel = jnp.exp(el - m).reshape(BR, 1)
    num = np_ref[0] + np_ref[1] + eel * h2_ref[...]
    den = (dp_ref[0, 0, 0, :] + dp_ref[1, 0, 0, :]).reshape(BR, 1) + eel
    h3_ref[...] = jnp.maximum(num / den + b2_ref[...].reshape(1, D), 0.0)


def _k5a_body(h3_ref, wr_ref, bl_ref, r_ref):
    r_ref[...] = (jnp.dot(h3_ref[...], wr_ref[...],
                          preferred_element_type=jnp.float32)
                  + bl_ref[...].reshape(1, D))


def _k5b_body(q_ref, cnt_ref, r_ref, wl_ref, out_ref):
    cnt = cnt_ref[0, 0, 0, :] + cnt_ref[1, 0, 0, :]
    agg = (q_ref[0] + q_ref[1]) / jnp.maximum(cnt, 1.0).reshape(BR, 1)
    out_ref[...] = (jnp.dot(agg, wl_ref[...], preferred_element_type=jnp.float32)
                    + r_ref[...])


def _row_spec():
    return pl.BlockSpec((BR, D), lambda i: (i, 0))


def _full_spec(shape):
    nd = len(shape)
    return pl.BlockSpec(shape, lambda i: (0,) * nd)


def _vec128_spec():
    return pl.BlockSpec((1, 1, 128), lambda i: (i, 0, 0))


@jax.jit
def _impl(x, edge_index, W1, b1, W2, att_src, att_dst, b2, Wl, bl, Wr):
    src = edge_index[0]
    dst = edge_index[1]
    src3 = src.reshape(NW, NCH, CH)
    dst3 = dst.reshape(NW, NCH, CH)
    srcg = src.reshape(NW, NG, GRP, CH2)
    dstg = dst.reshape(NW, NG, GRP, CH2)
    sd4 = jnp.concatenate([srcg, dstg], axis=2)
    xp = jnp.zeros((NP, D), jnp.float32).at[:N].set(x)
    z1 = jnp.zeros((NP,), jnp.float32)
    z2 = jnp.zeros((NP, D), jnp.float32)

    cntp = _sc_count(dst3, z1)
    cnt3 = cntp.reshape(NSC, NBLK, 1, BR)
    h = pl.pallas_call(
        _k0_body,
        grid=(NBLK,),
        in_specs=[_row_spec(), _full_spec((D, D))],
        out_specs=_row_spec(),
        out_shape=jax.ShapeDtypeStruct((NP, D), jnp.float32),
    )(xp, W1)

    g, dinvf = pl.pallas_call(
        _k1_body,
        grid=(NBLK,),
        in_specs=[_row_spec(),
                  pl.BlockSpec((NSC, 1, 1, BR), lambda i: (0, i, 0, 0))],
        out_specs=[_row_spec(), _vec128_spec()],
        out_shape=[jax.ShapeDtypeStruct((NP, D), jnp.float32),
                   jax.ShapeDtypeStruct((NBLK, 1, BR), jnp.float32)],
    )(h, cnt3)

    p = _sc_agg(sd4, g, z2)

    h2, als2, ald2, ms, md = pl.pallas_call(
        _k2_body,
        grid=(NBLK,),
        in_specs=[pl.BlockSpec((NSC, BR, D), lambda i: (0, i, 0)),
                  _row_spec(), _vec128_spec(), _full_spec((D,)),
                  _full_spec((D, D)), _full_spec((D,)), _full_spec((D,))],
        out_specs=[_row_spec(), _vec128_spec(), _vec128_spec(),
                   pl.BlockSpec((8, 128), lambda i: (0, 0)),
                   pl.BlockSpec((8, 128), lambda i: (0, 0))],
        out_shape=[jax.ShapeDtypeStruct((NP, D), jnp.float32),
                   jax.ShapeDtypeStruct((NBLK, 1, BR), jnp.float32),
                   jax.ShapeDtypeStruct((NBLK, 1, BR), jnp.float32),
                   jax.ShapeDtypeStruct((8, 128), jnp.float32),
                   jax.ShapeDtypeStruct((8, 128), jnp.float32)],
    )(p, g, dinvf, b1, W2, att_src, att_dst)

    dp, ee = _sc_eegen(src3, dst3, als2.reshape(NP), ald2.reshape(NP), ms, md, z1)
    eeg4 = jax.lax.bitcast_convert_type(ee, jnp.int32).reshape(NW, NG, GRP, CH2)
    sde4 = jnp.concatenate([srcg, dstg, eeg4], axis=2)

    nump = _sc_gatagg(sde4, h2, z2)

    h3 = pl.pallas_call(
        _k4_body,
        grid=(NBLK,),
        in_specs=[pl.BlockSpec((NSC, BR, D), lambda i: (0, i, 0)),
                  pl.BlockSpec((NSC, 1, 1, BR), lambda i: (0, i, 0, 0)),
                  _row_spec(), _vec128_spec(), _vec128_spec(),
                  pl.BlockSpec((8, 128), lambda i: (0, 0)),
                  pl.BlockSpec((8, 128), lambda i: (0, 0)),
                  _full_spec((D,))],
        out_specs=_row_spec(),
        out_shape=jax.ShapeDtypeStruct((NP, D), jnp.float32),
    )(nump, dp.reshape(NSC, NBLK, 1, BR), h2, als2, ald2, ms, md, b2)

    q = _sc_agg(sd4, h3, z2)
    r = pl.pallas_call(
        _k5a_body,
        grid=(NBLK,),
        in_specs=[_row_spec(), _full_spec((D, D)), _full_spec((D,))],
        out_specs=_row_spec(),
        out_shape=jax.ShapeDtypeStruct((NP, D), jnp.float32),
    )(h3, Wr, bl)

    out = pl.pallas_call(
        _k5b_body,
        grid=(NBLK,),
        in_specs=[pl.BlockSpec((NSC, BR, D), lambda i: (0, i, 0)),
                  pl.BlockSpec((NSC, 1, 1, BR), lambda i: (0, i, 0, 0)),
                  _row_spec(), _full_spec((D, D))],
        out_specs=_row_spec(),
        out_shape=jax.ShapeDtypeStruct((NP, D), jnp.float32),
    )(q, cnt3, r, Wl)

    return out[:N]


def kernel(x, edge_index, W1, b1, W2, att_src, att_dst, b2, Wl, bl, Wr):
    return _impl(x, edge_index, W1, b1, W2, att_src, att_dst, b2, Wl, bl, Wr)


# speedup vs baseline: 1.0937x; 1.0937x over previous
import functools

import jax
import jax.numpy as jnp
from jax import lax
from jax.experimental import pallas as pl
from jax.experimental.pallas import tpu as pltpu
from jax.experimental.pallas import tpu_sc as plsc

N = 10000
E = 320000
D = 128
NP = 10240
BR = 128
NBLK = NP // BR
NSC = 2
NSUB = 16
NW = NSC * NSUB
ZPT = NP // NSUB
EPT = E // NW

CH = 80
NCH = EPT // CH
CH2 = 40
GRP = 10
NG = EPT // (CH2 * GRP)
NB = 4

_mesh = plsc.VectorSubcoreMesh(core_axis_name="c", subcore_axis_name="s")
_sc_params = pltpu.CompilerParams(needs_layout_passes=False)


@functools.partial(
    pl.kernel,
    out_type=jax.ShapeDtypeStruct((NSC, NP), jnp.float32),
    mesh=_mesh,
    scratch_types=[
        pltpu.VMEM((NCH, CH), jnp.int32),
        pltpu.VMEM((CH,), jnp.float32),
        pltpu.VMEM_SHARED((NP,), jnp.float32),
        pltpu.SemaphoreType.DMA,
    ],
    compiler_params=_sc_params,
)
def _sc_count(dst3_hbm, z1_hbm, out_hbm, didx, ones, acc, ssem):
    cid = lax.axis_index("c")
    sid = lax.axis_index("s")
    wid = cid * NSUB + sid
    for k in range(CH // 16):
        ones[pl.ds(16 * k, 16)] = jnp.full((16,), 1.0, jnp.float32)
    pltpu.sync_copy(z1_hbm.at[pl.ds(sid * ZPT, ZPT)], acc.at[pl.ds(sid * ZPT, ZPT)])
    pltpu.sync_copy(dst3_hbm.at[wid], didx)
    plsc.subcore_barrier()

    @pl.loop(0, NCH)
    def _(j):
        pltpu.async_copy(ones, acc.at[didx.at[j]], ssem, add=True)

    @pl.loop(0, NCH)
    def _(j):
        pltpu.make_async_copy(ones, acc.at[didx.at[j]], ssem).wait()

    plsc.subcore_barrier()
    pltpu.sync_copy(acc.at[pl.ds(sid * ZPT, ZPT)],
                    out_hbm.at[cid, pl.ds(sid * ZPT, ZPT)])


@functools.partial(
    pl.kernel,
    out_type=jax.ShapeDtypeStruct((NSC, NP, D), jnp.float32),
    mesh=_mesh,
    scratch_types=[
        pltpu.VMEM((2 * GRP, CH2), jnp.int32),
        pltpu.VMEM((2 * GRP, CH2), jnp.int32),
        pltpu.VMEM((CH2, D), jnp.float32),
        pltpu.VMEM((CH2, D), jnp.float32),
        pltpu.VMEM((CH2, D), jnp.float32),
        pltpu.VMEM((CH2, D), jnp.float32),
        pltpu.VMEM_SHARED((NP, D), jnp.float32),
    ] + [pltpu.SemaphoreType.DMA] * 10,
    compiler_params=_sc_params,
)
def _sc_agg(sd4_hbm, g_hbm, z2_hbm, out_hbm,
            idxg0, idxg1, rows0, rows1, rows2, rows3, acc,
            gsem0, gsem1, gsem2, gsem3, ssem0, ssem1, ssem2, ssem3,
            isem0, isem1):
    rows = (rows0, rows1, rows2, rows3)
    gsem = (gsem0, gsem1, gsem2, gsem3)
    ssem = (ssem0, ssem1, ssem2, ssem3)
    cid = lax.axis_index("c")
    sid = lax.axis_index("s")
    wid = cid * NSUB + sid
    pltpu.sync_copy(z2_hbm.at[pl.ds(sid * ZPT, ZPT)], acc.at[pl.ds(sid * ZPT, ZPT)])
    pltpu.async_copy(sd4_hbm.at[wid, 0], idxg0, isem0)
    plsc.subcore_barrier()

    def process(idxg):
        gd = {}
        sd = {}
        for i in range(NB):
            gd[i] = pltpu.async_copy(g_hbm.at[idxg.at[i]], rows[i % NB],
                                     gsem[i % NB])
        for i in range(GRP):
            gd[i].wait()
            sd[i] = pltpu.async_copy(rows[i % NB], acc.at[idxg.at[GRP + i]],
                                     ssem[i % NB], add=True)
            if i + NB < GRP:
                sd[i].wait()
                gd[i + NB] = pltpu.async_copy(g_hbm.at[idxg.at[i + NB]],
                                              rows[i % NB], gsem[i % NB])
        for i in range(GRP - NB, GRP):
            sd[i].wait()

    @pl.loop(0, NG // 2)
    def _(m):
        g0 = 2 * m
        st1 = pltpu.async_copy(sd4_hbm.at[wid, g0 + 1], idxg1, isem1)
        pltpu.make_async_copy(sd4_hbm.at[wid, g0], idxg0, isem0).wait()
        process(idxg0)
        pltpu.async_copy(sd4_hbm.at[wid, g0 + 2], idxg0, isem0)
        st1.wait()
        process(idxg1)

    pltpu.make_async_copy(sd4_hbm.at[wid, NG - 1], idxg0, isem0).wait()
    process(idxg0)

    plsc.subcore_barrier()
    pltpu.sync_copy(acc.at[pl.ds(sid * ZPT, ZPT)],
                    out_hbm.at[cid, pl.ds(sid * ZPT, ZPT)])


@functools.partial(
    pl.kernel,
    out_type=(jax.ShapeDtypeStruct((NSC, NP), jnp.float32),
              jax.ShapeDtypeStruct((E,), jnp.float32)),
    mesh=_mesh,
    scratch_types=[
        pltpu.VMEM((NCH, CH), jnp.int32),
        pltpu.VMEM((NCH, CH), jnp.int32),
        pltpu.VMEM((EPT,), jnp.float32),
        pltpu.VMEM((NP,), jnp.float32),
        pltpu.VMEM((NP,), jnp.float32),
        pltpu.VMEM((16,), jnp.float32),
        pltpu.VMEM((16,), jnp.float32),
        pltpu.VMEM_SHARED((NP,), jnp.float32),
        pltpu.SemaphoreType.DMA,
    ],
    compiler_params=_sc_params,
)
def _sc_eegen(src3_hbm, dst3_hbm, als_hbm, ald_hbm, ms_hbm, md_hbm, z1_hbm,
              dp_hbm, ee_hbm,
              sidx, didx, eeall, als_v, ald_v, msv, mdv, dacc, ssem):
    cid = lax.axis_index("c")
    sid = lax.axis_index("s")
    wid = cid * NSUB + sid
    pltpu.sync_copy(als_hbm, als_v)
    pltpu.sync_copy(ald_hbm, ald_v)
    pltpu.sync_copy(ms_hbm.at[0, pl.ds(0, 16)], msv)
    pltpu.sync_copy(md_hbm.at[0, pl.ds(0, 16)], mdv)
    pltpu.sync_copy(z1_hbm.at[pl.ds(sid * ZPT, ZPT)], dacc.at[pl.ds(sid * ZPT, ZPT)])
    pltpu.sync_copy(src3_hbm.at[wid], sidx)
    pltpu.sync_copy(dst3_hbm.at[wid], didx)
    plsc.subcore_barrier()
    mvec = jnp.maximum(msv[...] + mdv[...], 0.0)

    @pl.loop(0, NCH)
    def _(j):
        @pl.loop(0, CH // 16)
        def _(k):
            s16 = sidx[j, pl.ds(16 * k, 16)]
            d16 = didx[j, pl.ds(16 * k, 16)]
            e = plsc.load_gather(als_v, [s16]) + plsc.load_gather(ald_v, [d16])
            e = jnp.where(e > 0, e, 0.2 * e)
            eeall[pl.ds(j * CH + 16 * k, 16)] = jnp.exp(e - mvec)

        pltpu.async_copy(eeall.at[pl.ds(j * CH, CH)], dacc.at[didx.at[j]],
                         ssem, add=True)

    @pl.loop(0, NCH)
    def _(j):
        pltpu.make_async_copy(eeall.at[pl.ds(j * CH, CH)], dacc.at[didx.at[j]],
                              ssem).wait()

    plsc.subcore_barrier()
    pltpu.sync_copy(dacc.at[pl.ds(sid * ZPT, ZPT)],
                    dp_hbm.at[cid, pl.ds(sid * ZPT, ZPT)])
    pltpu.sync_copy(eeall, ee_hbm.at[pl.ds(wid * EPT, EPT)])


@functools.partial(
    pl.kernel,
    out_type=jax.ShapeDtypeStruct((NSC, NP, D), jnp.float32),
    mesh=_mesh,
    scratch_types=[
        pltpu.VMEM((3 * GRP, CH2), jnp.int32),
        pltpu.VMEM((3 * GRP, CH2), jnp.int32),
        pltpu.VMEM((CH2, D), jnp.float32),
        pltpu.VMEM((CH2, D), jnp.float32),
        pltpu.VMEM((CH2, D), jnp.float32),
        pltpu.VMEM((CH2, D), jnp.float32),
        pltpu.VMEM_SHARED((NP, D), jnp.float32),
    ] + [pltpu.SemaphoreType.DMA] * 10,
    compiler_params=_sc_params,
)
def _sc_gatagg(sde4_hbm, h2_hbm, z2_hbm, out_hbm,
               idxg0, idxg1, rows0, rows1, rows2, rows3, acc,
               gsem0, gsem1, gsem2, gsem3, ssem0, ssem1, ssem2, ssem3,
               isem0, isem1):
    rows = (rows0, rows1, rows2, rows3)
    gsem = (gsem0, gsem1, gsem2, gsem3)
    ssem = (ssem0, ssem1, ssem2, ssem3)
    cid = lax.axis_index("c")
    sid = lax.axis_index("s")
    wid = cid * NSUB + sid
    pltpu.sync_copy(z2_hbm.at[pl.ds(sid * ZPT, ZPT)], acc.at[pl.ds(sid * ZPT, ZPT)])
    pltpu.async_copy(sde4_hbm.at[wid, 0], idxg0, isem0)
    plsc.subcore_barrier()

    def scale(idxg, i):
        i16 = jnp.full((16,), 2 * GRP + i, jnp.int32)
        one16 = jnp.full((16,), 1, jnp.int32)

        @pl.loop(0, CH2)
        def _(r):
            w = plsc.bitcast(plsc.load_gather(idxg, [i16, one16 * r]),
                             jnp.float32)
            for c in range(D // 16):
                rows[i % NB][r, pl.ds(16 * c, 16)] = (
                    rows[i % NB][r, pl.ds(16 * c, 16)] * w)

    def process(idxg):
        gd = {}
        sd = {}
        for i in range(NB):
            gd[i] = pltpu.async_copy(h2_hbm.at[idxg.at[i]], rows[i % NB],
                                     gsem[i % NB])
        for i in range(GRP):
            gd[i].wait()
            scale(idxg, i)
            sd[i] = pltpu.async_copy(rows[i % NB], acc.at[idxg.at[GRP + i]],
                                     ssem[i % NB], add=True)
            if i + NB < GRP:
                sd[i].wait()
                gd[i + NB] = pltpu.async_copy(h2_hbm.at[idxg.at[i + NB]],
                                              rows[i % NB], gsem[i % NB])
        for i in range(GRP - NB, GRP):
            sd[i].wait()

    @pl.loop(0, NG // 2)
    def _(m):
        g0 = 2 * m
        st1 = pltpu.async_copy(sde4_hbm.at[wid, g0 + 1], idxg1, isem1)
        pltpu.make_async_copy(sde4_hbm.at[wid, g0], idxg0, isem0).wait()
        process(idxg0)
        pltpu.async_copy(sde4_hbm.at[wid, g0 + 2], idxg0, isem0)
        st1.wait()
        process(idxg1)

    pltpu.make_async_copy(sde4_hbm.at[wid, NG - 1], idxg0, isem0).wait()
    process(idxg0)

    plsc.subcore_barrier()
    pltpu.sync_copy(acc.at[pl.ds(sid * ZPT, ZPT)],
                    out_hbm.at[cid, pl.ds(sid * ZPT, ZPT)])


def _k1_body(x_ref, w_ref, cnt_ref, g_ref, dinv_ref):
    cnt = cnt_ref[0, 0, 0, :] + cnt_ref[1, 0, 0, :]
    dinv = lax.rsqrt(cnt + 1.0)
    h = jnp.dot(x_ref[...], w_ref[...], preferred_element_type=jnp.float32)
    g_ref[...] = h * dinv.reshape(BR, 1)
    dinv_ref[...] = dinv.reshape(1, 1, BR)


def _k2_body(p_ref, g_ref, dinv_ref, b1_ref, w2_ref, as_ref, ad_ref,
             h2_ref, als_ref, ald_ref, ms_ref, md_ref):
    i = pl.program_id(0)
    dinv = dinv_ref[0, 0, :].reshape(BR, 1)
    sig = p_ref[0] + p_ref[1] + g_ref[...]
    h1 = jnp.maximum(dinv * sig + b1_ref[...].reshape(1, D), 0.0)
    h2 = jnp.dot(h1, w2_ref[...], preferred_element_type=jnp.float32)
    h2_ref[...] = h2
    als = jnp.sum(h2 * as_ref[...].reshape(1, D), axis=1)
    ald = jnp.sum(h2 * ad_ref[...].reshape(1, D), axis=1)
    als_ref[...] = als.reshape(1, 1, BR)
    ald_ref[...] = ald.reshape(1, 1, BR)
    bs = jnp.max(als)
    bd = jnp.max(ald)

    @pl.when(i == 0)
    def _():
        ms_ref[...] = jnp.full((8, 128), bs, jnp.float32)
        md_ref[...] = jnp.full((8, 128), bd, jnp.float32)

    @pl.when(i > 0)
    def _():
        ms_ref[...] = jnp.maximum(ms_ref[...], bs)
        md_ref[...] = jnp.maximum(md_ref[...], bd)


def _k4_body(np_ref, dp_ref, h2_ref, als_ref, ald_ref, ms_ref, md_ref,
             b2_ref, h3_ref):
    m = jnp.maximum(ms_ref[0, 0] + md_ref[0, 0], 0.0)
    al = als_ref[0, 0, :] + ald_ref[0, 0, :]
    el = jnp.where(al > 0, al, 0.2 * al)
    eel = jnp.exp(el - m).reshape(BR, 1)
    num = np_ref[0] + np_ref[1] + eel * h2_ref[...]
    den = (dp_ref[0, 0, 0, :] + dp_ref[1, 0, 0, :]).reshape(BR, 1) + eel
    h3_ref[...] = jnp.maximum(num / den + b2_ref[...].reshape(1, D), 0.0)


def _k5_body(q_ref, cnt_ref, h3_ref, wl_ref, bl_ref, wr_ref, out_ref):
    cnt = cnt_ref[0, 0, 0, :] + cnt_ref[1, 0, 0, :]
    agg = (q_ref[0] + q_ref[1]) / jnp.maximum(cnt, 1.0).reshape(BR, 1)
    out_ref[...] = (jnp.dot(agg, wl_ref[...], preferred_element_type=jnp.float32)
                    + bl_ref[...].reshape(1, D)
                    + jnp.dot(h3_ref[...], wr_ref[...], preferred_element_type=jnp.float32))


def _row_spec():
    return pl.BlockSpec((BR, D), lambda i: (i, 0))


def _full_spec(shape):
    nd = len(shape)
    return pl.BlockSpec(shape, lambda i: (0,) * nd)


def _vec128_spec():
    return pl.BlockSpec((1, 1, 128), lambda i: (i, 0, 0))


@jax.jit
def _impl(x, edge_index, W1, b1, W2, att_src, att_dst, b2, Wl, bl, Wr):
    src = edge_index[0]
    dst = edge_index[1]
    src3 = src.reshape(NW, NCH, CH)
    dst3 = dst.reshape(NW, NCH, CH)
    srcg = src.reshape(NW, NG, GRP, CH2)
    dstg = dst.reshape(NW, NG, GRP, CH2)
    sd4 = jnp.concatenate([srcg, dstg], axis=2)
    xp = jnp.zeros((NP, D), jnp.float32).at[:N].set(x)
    z1 = jnp.zeros((NP,), jnp.float32)
    z2 = jnp.zeros((NP, D), jnp.float32)

    cntp = _sc_count(dst3, z1)
    cnt3 = cntp.reshape(NSC, NBLK, 1, BR)

    g, dinvf = pl.pallas_call(
        _k1_body,
        grid=(NBLK,),
        in_specs=[_row_spec(), _full_spec((D, D)),
                  pl.BlockSpec((NSC, 1, 1, BR), lambda i: (0, i, 0, 0))],
        out_specs=[_row_spec(), _vec128_spec()],
        out_shape=[jax.ShapeDtypeStruct((NP, D), jnp.float32),
                   jax.ShapeDtypeStruct((NBLK, 1, BR), jnp.float32)],
    )(xp, W1, cnt3)

    p = _sc_agg(sd4, g, z2)

    h2, als2, ald2, ms, md = pl.pallas_call(
        _k2_body,
        grid=(NBLK,),
        in_specs=[pl.BlockSpec((NSC, BR, D), lambda i: (0, i, 0)),
                  _row_spec(), _vec128_spec(), _full_spec((D,)),
                  _full_spec((D, D)), _full_spec((D,)), _full_spec((D,))],
        out_specs=[_row_spec(), _vec128_spec(), _vec128_spec(),
                   pl.BlockSpec((8, 128), lambda i: (0, 0)),
                   pl.BlockSpec((8, 128), lambda i: (0, 0))],
        out_shape=[jax.ShapeDtypeStruct((NP, D), jnp.float32),
                   jax.ShapeDtypeStruct((NBLK, 1, BR), jnp.float32),
                   jax.ShapeDtypeStruct((NBLK, 1, BR), jnp.float32),
                   jax.ShapeDtypeStruct((8, 128), jnp.float32),
                   jax.ShapeDtypeStruct((8, 128), jnp.float32)],
    )(p, g, dinvf, b1, W2, att_src, att_dst)

    dp, ee = _sc_eegen(src3, dst3, als2.reshape(NP), ald2.reshape(NP), ms, md, z1)
    eeg4 = jax.lax.bitcast_convert_type(ee, jnp.int32).reshape(NW, NG, GRP, CH2)
    sde4 = jnp.concatenate([srcg, dstg, eeg4], axis=2)

    nump = _sc_gatagg(sde4, h2, z2)

    h3 = pl.pallas_call(
        _k4_body,
        grid=(NBLK,),
        in_specs=[pl.BlockSpec((NSC, BR, D), lambda i: (0, i, 0)),
                  pl.BlockSpec((NSC, 1, 1, BR), lambda i: (0, i, 0, 0)),
                  _row_spec(), _vec128_spec(), _vec128_spec(),
                  pl.BlockSpec((8, 128), lambda i: (0, 0)),
                  pl.BlockSpec((8, 128), lambda i: (0, 0)),
                  _full_spec((D,))],
        out_specs=_row_spec(),
        out_shape=jax.ShapeDtypeStruct((NP, D), jnp.float32),
    )(nump, dp.reshape(NSC, NBLK, 1, BR), h2, als2, ald2, ms, md, b2)

    q = _sc_agg(sd4, h3, z2)

    out = pl.pallas_call(
        _k5_body,
        grid=(NBLK,),
        in_specs=[pl.BlockSpec((NSC, BR, D), lambda i: (0, i, 0)),
                  pl.BlockSpec((NSC, 1, 1, BR), lambda i: (0, i, 0, 0)),
                  _row_spec(), _full_spec((D, D)), _full_spec((D,)),
                  _full_spec((D, D))],
        out_specs=_row_spec(),
        out_shape=jax.ShapeDtypeStruct((NP, D), jnp.float32),
    )(q, cnt3, h3, Wl, bl, Wr)

    return out[:N]


def kernel(x, edge_index, W1, b1, W2, att_src, att_dst, b2, Wl, bl, Wr):
    return _impl(x, edge_index, W1, b1, W2, att_src, att_dst, b2, Wl, bl, Wr)
